# Initial kernel scaffold; baseline (speedup 1.0000x reference)
import functools

import jax
import jax.numpy as jnp
from jax import lax
from jax.experimental import pallas as pl
from jax.experimental.pallas import tpu as pltpu
from jax.experimental.pallas import tpu_sc as plsc

_N = 10000
_DEG = 32
_FEAT = 128
_NW = 32
_NPW = 320
_NPAD = _NW * _NPW
_EPB = 128
_GPB = _EPB // _DEG
_NBLK = _NPW // _GPB

_mesh = plsc.VectorSubcoreMesh(core_axis_name="c", subcore_axis_name="s")


def _rsqrt_vec(v):
    bits = plsc.bitcast(v, jnp.int32)
    y = plsc.bitcast(jnp.int32(0x5F3759DF) - (bits >> 1), jnp.float32)
    for _ in range(3):
        y = y * (1.5 - 0.5 * v * y * y)
    return y


@functools.partial(
    pl.kernel,
    out_type=(
        jax.ShapeDtypeStruct((_NPAD,), jnp.float32),
        jax.ShapeDtypeStruct((_NPAD,), jnp.float32),
    ),
    mesh=_mesh,
    scratch_types=[
        pltpu.VMEM((16, _FEAT), jnp.float32),
        pltpu.VMEM((16,), jnp.float32),
        pltpu.VMEM((16,), jnp.float32),
        pltpu.VMEM((16,), jnp.float32),
        pltpu.VMEM((16,), jnp.float32),
    ],
)
def _norms(x_hbm, beta_hbm, inv_hbm, binv_hbm, xbuf, ssbuf, betabuf, invb, binvb):
    wid = lax.axis_index("s") * 2 + lax.axis_index("c")
    pltpu.sync_copy(beta_hbm, betabuf)
    beta = betabuf[0]

    def grp(g, carry):
        row0 = wid * _NPW + g * 16
        pltpu.sync_copy(x_hbm.at[pl.ds(row0, 16)], xbuf)
        for i in range(16):
            v = xbuf[i, pl.ds(0, 16)]
            acc = v * v
            for r in range(1, 8):
                v = xbuf[i, pl.ds(r * 16, 16)]
                acc = acc + v * v
            ssbuf[i] = jnp.sum(acc)
        ss = ssbuf[...]
        norm = ss * _rsqrt_vec(ss)
        inv = 1.0 / (norm + 1e-12)
        invb[...] = inv
        binvb[...] = inv * beta
        pltpu.sync_copy(invb, inv_hbm.at[pl.ds(row0, 16)])
        pltpu.sync_copy(binvb, binv_hbm.at[pl.ds(row0, 16)])
        return carry

    lax.fori_loop(0, _NPW // 16, grp, 0)


@functools.partial(
    pl.kernel,
    out_type=jax.ShapeDtypeStruct((_NPAD, _FEAT), jnp.float32),
    mesh=_mesh,
    scratch_types=[
        pltpu.VMEM((_NBLK, _EPB), jnp.int32),
        pltpu.VMEM((_NPAD,), jnp.float32),
        pltpu.VMEM((_NPAD,), jnp.float32),
        pltpu.VMEM((_GPB, _FEAT), jnp.float32),
        pltpu.VMEM((_EPB, _FEAT), jnp.float32),
        pltpu.VMEM((_DEG,), jnp.float32),
        pltpu.VMEM((_GPB, _FEAT), jnp.float32),
        pltpu.SemaphoreType.DMA,
    ],
)
def _agnn(x_hbm, col_hbm, inv_hbm, binv_hbm, out_hbm,
          colv, invv, binvv, xi, rows, attn, outb, gsem):
    wid = lax.axis_index("s") * 2 + lax.axis_index("c")
    pltpu.sync_copy(col_hbm.at[wid], colv)
    pltpu.sync_copy(inv_hbm, invv)
    pltpu.sync_copy(binv_hbm, binvv)

    def block(b, carry):
        node0 = wid * _NPW + b * _GPB
        pltpu.sync_copy(x_hbm.at[pl.ds(node0, _GPB)], xi)
        pltpu.async_copy(x_hbm.at[colv.at[b]], rows, gsem).wait()

        def node(n, c2):
            nb = n * _DEG
            xr = [xi[n, pl.ds(r * 16, 16)] for r in range(8)]
            for j in range(_DEG):
                rr = nb + j
                acc = xr[0] * rows[rr, pl.ds(0, 16)]
                for r in range(1, 8):
                    acc = acc + xr[r] * rows[rr, pl.ds(r * 16, 16)]
                attn[j] = jnp.sum(acc)
            bi = binvv[node0 + n]
            sc = []
            for g in range(2):
                d = attn[pl.ds(g * 16, 16)]
                jidx = colv[b, pl.ds(nb + g * 16, 16)]
                invj = plsc.load_gather(invv, [jidx])
                sc.append(d * invj * bi)
            m = jnp.max(jnp.maximum(sc[0], sc[1]))
            e0 = jnp.exp(sc[0] - m)
            e1 = jnp.exp(sc[1] - m)
            rden = 1.0 / (jnp.sum(e0 + e1) + 1e-16)
            attn[pl.ds(0, 16)] = e0 * rden
            attn[pl.ds(16, 16)] = e1 * rden
            ao = [jnp.zeros((16,), jnp.float32) for _ in range(8)]
            for j in range(_DEG):
                a = attn[j]
                rr = nb + j
                for r in range(8):
                    ao[r] = ao[r] + a * rows[rr, pl.ds(r * 16, 16)]
            for r in range(8):
                outb[n, pl.ds(r * 16, 16)] = ao[r]
            return c2

        lax.fori_loop(0, _GPB, node, 0)
        pltpu.sync_copy(outb, out_hbm.at[pl.ds(node0, _GPB)])
        return carry

    lax.fori_loop(0, _NBLK, block, 0)


def kernel(x, row_id, row_ptr, col_id, beta):
    del row_id, row_ptr
    xp = jnp.zeros((_NPAD, _FEAT), jnp.float32).at[:_N].set(x)
    colp = jnp.zeros((_NPAD * _DEG,), jnp.int32).at[: _N * _DEG].set(col_id)
    col3 = colp.reshape(_NW, _NBLK, _EPB)
    beta16 = jnp.zeros((16,), jnp.float32).at[0].set(beta[0])
    inv, binv = _norms(xp, beta16)
    out = _agnn(xp, col3, inv, binv)
    return out[:_N]


# speedup vs baseline: 9.7283x; 9.7283x over previous
import functools

import jax
import jax.numpy as jnp
from jax import lax
from jax.experimental import pallas as pl
from jax.experimental.pallas import tpu as pltpu
from jax.experimental.pallas import tpu_sc as plsc

_N = 10000
_DEG = 32
_FEAT = 128
_NW = 32
_NPW = 320
_NPAD = _NW * _NPW
_EPB = 128
_GPB = _EPB // _DEG
_NBLK = _NPW // _GPB

_mesh = plsc.VectorSubcoreMesh(core_axis_name="c", subcore_axis_name="s")
_cparams = pltpu.CompilerParams(needs_layout_passes=False)


def _rsqrt_vec(v):
    bits = plsc.bitcast(v, jnp.int32)
    y = plsc.bitcast(jnp.int32(0x5F3759DF) - (bits >> 1), jnp.float32)
    for _ in range(3):
        y = y * (1.5 - 0.5 * v * y * y)
    return y


def _splat(s, dtype=jnp.float32):
    return jnp.full((16,), s, dtype)


@functools.partial(
    pl.kernel,
    out_type=(
        jax.ShapeDtypeStruct((_NPAD,), jnp.float32),
        jax.ShapeDtypeStruct((_NPAD,), jnp.float32),
    ),
    mesh=_mesh,
    compiler_params=_cparams,
    scratch_types=[
        pltpu.VMEM((16, _FEAT), jnp.float32),
        pltpu.VMEM((16,), jnp.float32),
        pltpu.VMEM((16,), jnp.float32),
        pltpu.VMEM((16,), jnp.float32),
    ],
)
def _norms(x_hbm, beta_hbm, inv_hbm, binv_hbm, xbuf, betabuf, invb, binvb):
    wid = lax.axis_index("s") * 2 + lax.axis_index("c")
    pltpu.sync_copy(beta_hbm, betabuf)
    beta = betabuf[...][0]
    lanes = lax.iota(jnp.int32, 16)

    def grp(g, carry):
        row0 = wid * _NPW + g * 16
        pltpu.sync_copy(x_hbm.at[pl.ds(row0, 16)], xbuf)
        ss = jnp.zeros((16,), jnp.float32)
        for i in range(16):
            v = xbuf[i, pl.ds(0, 16)]
            acc = v * v
            for r in range(1, 8):
                v = xbuf[i, pl.ds(r * 16, 16)]
                acc = acc + v * v
            ss = jnp.where(lanes == i, _splat(jnp.sum(acc)), ss)
        norm = ss * _rsqrt_vec(ss)
        inv = 1.0 / (norm + 1e-12)
        invb[...] = inv
        binvb[...] = inv * _splat(beta)
        pltpu.sync_copy(invb, inv_hbm.at[pl.ds(row0, 16)])
        pltpu.sync_copy(binvb, binv_hbm.at[pl.ds(row0, 16)])
        return carry

    lax.fori_loop(0, _NPW // 16, grp, 0)


@functools.partial(
    pl.kernel,
    out_type=jax.ShapeDtypeStruct((_NPAD, _FEAT), jnp.float32),
    mesh=_mesh,
    compiler_params=_cparams,
    scratch_types=[
        pltpu.VMEM((_NBLK, _EPB), jnp.int32),
        pltpu.VMEM((_NPAD,), jnp.float32),
        pltpu.VMEM((_NPAD,), jnp.float32),
        pltpu.VMEM((_GPB, _FEAT), jnp.float32),
        pltpu.VMEM((_EPB, _FEAT), jnp.float32),
        pltpu.VMEM((_GPB, _FEAT), jnp.float32),
        pltpu.SemaphoreType.DMA,
    ],
)
def _agnn(x_hbm, col_hbm, inv_hbm, binv_hbm, out_hbm,
          colv, invv, binvv, xi, rows, outb, gsem):
    wid = lax.axis_index("s") * 2 + lax.axis_index("c")
    pltpu.sync_copy(col_hbm.at[wid], colv)
    pltpu.sync_copy(inv_hbm, invv)
    pltpu.sync_copy(binv_hbm, binvv)
    lanes = lax.iota(jnp.int32, 16)

    def block(b, carry):
        node0 = wid * _NPW + b * _GPB
        pltpu.sync_copy(x_hbm.at[pl.ds(node0, _GPB)], xi)
        pltpu.async_copy(x_hbm.at[colv.at[b]], rows, gsem).wait()

        def node(n, c2):
            nb = n * _DEG
            xr = [xi[n, pl.ds(r * 16, 16)] for r in range(8)]
            dots = []
            for g in range(2):
                d = jnp.zeros((16,), jnp.float32)
                for jj in range(16):
                    rr = nb + g * 16 + jj
                    acc = xr[0] * rows[rr, pl.ds(0, 16)]
                    for r in range(1, 8):
                        acc = acc + xr[r] * rows[rr, pl.ds(r * 16, 16)]
                    d = jnp.where(lanes == jj, _splat(jnp.sum(acc)), d)
                dots.append(d)
            bi = plsc.load_gather(binvv, [_splat(node0 + n, jnp.int32)])
            sc = []
            for g in range(2):
                jidx = colv[b, pl.ds(nb + g * 16, 16)]
                invj = plsc.load_gather(invv, [jidx])
                sc.append(dots[g] * invj * bi)
            m = _splat(jnp.max(jnp.maximum(sc[0], sc[1])))
            e0 = jnp.exp(sc[0] - m)
            e1 = jnp.exp(sc[1] - m)
            rden = 1.0 / (_splat(jnp.sum(e0 + e1)) + 1e-16)
            a0 = e0 * rden
            a1 = e1 * rden
            ao = [jnp.zeros((16,), jnp.float32) for _ in range(8)]
            for g, av in enumerate((a0, a1)):
                for jj in range(16):
                    a = av[jj]
                    rr = nb + g * 16 + jj
                    for r in range(8):
                        ao[r] = ao[r] + a * rows[rr, pl.ds(r * 16, 16)]
            for r in range(8):
                outb[n, pl.ds(r * 16, 16)] = ao[r]
            return c2

        lax.fori_loop(0, _GPB, node, 0)
        pltpu.sync_copy(outb, out_hbm.at[pl.ds(node0, _GPB)])
        return carry

    lax.fori_loop(0, _NBLK, block, 0)


def kernel(x, row_id, row_ptr, col_id, beta):
    del row_id, row_ptr
    xp = jnp.zeros((_NPAD, _FEAT), jnp.float32).at[:_N].set(x)
    colp = jnp.zeros((_NPAD * _DEG,), jnp.int32).at[: _N * _DEG].set(col_id)
    col3 = colp.reshape(_NW, _NBLK, _EPB)
    beta16 = jnp.zeros((16,), jnp.float32).at[0].set(beta[0])
    inv, binv = _norms(xp, beta16)
    out = _agnn(xp, col3, inv, binv)
    return out[:_N]


# speedup vs baseline: 12.3254x; 1.2670x over previous
import functools

import jax
import jax.numpy as jnp
from jax import lax
from jax.experimental import pallas as pl
from jax.experimental.pallas import tpu as pltpu
from jax.experimental.pallas import tpu_sc as plsc

_N = 10000
_DEG = 32
_FEAT = 128
_NW = 32
_NPW = 320
_NPAD = _NW * _NPW
_EPB = 128
_GPB = _EPB // _DEG
_NBLK = _NPW // _GPB

_mesh = plsc.VectorSubcoreMesh(core_axis_name="c", subcore_axis_name="s")
_cparams = pltpu.CompilerParams(needs_layout_passes=False)


def _rsqrt_vec(v):
    bits = plsc.bitcast(v, jnp.int32)
    y = plsc.bitcast(jnp.int32(0x5F3759DF) - (bits >> 1), jnp.float32)
    for _ in range(3):
        y = y * (1.5 - 0.5 * v * y * y)
    return y


def _splat(s, dtype=jnp.float32):
    return jnp.full((16,), s, dtype)


@functools.partial(
    pl.kernel,
    out_type=(
        jax.ShapeDtypeStruct((_NPAD,), jnp.float32),
        jax.ShapeDtypeStruct((_NPAD,), jnp.float32),
    ),
    mesh=_mesh,
    compiler_params=_cparams,
    scratch_types=[
        pltpu.VMEM((16, _FEAT), jnp.float32),
        pltpu.VMEM((16,), jnp.float32),
        pltpu.VMEM((16,), jnp.float32),
        pltpu.VMEM((16,), jnp.float32),
    ],
)
def _norms(x_hbm, beta_hbm, inv_hbm, binv_hbm, xbuf, betabuf, invb, binvb):
    wid = lax.axis_index("s") * 2 + lax.axis_index("c")
    pltpu.sync_copy(beta_hbm, betabuf)
    beta = betabuf[...][0]
    lanes = lax.iota(jnp.int32, 16)

    def grp(g, carry):
        row0 = wid * _NPW + g * 16
        pltpu.sync_copy(x_hbm.at[pl.ds(row0, 16)], xbuf)
        ss = jnp.zeros((16,), jnp.float32)
        for i in range(16):
            v = xbuf[i, pl.ds(0, 16)]
            acc = v * v
            for r in range(1, 8):
                v = xbuf[i, pl.ds(r * 16, 16)]
                acc = acc + v * v
            ss = jnp.where(lanes == i, _splat(jnp.sum(acc)), ss)
        norm = ss * _rsqrt_vec(ss)
        inv = 1.0 / (norm + 1e-12)
        invb[...] = inv
        binvb[...] = inv * _splat(beta)
        pltpu.sync_copy(invb, inv_hbm.at[pl.ds(row0, 16)])
        pltpu.sync_copy(binvb, binv_hbm.at[pl.ds(row0, 16)])
        return carry

    lax.fori_loop(0, _NPW // 16, grp, 0)


@functools.partial(
    pl.kernel,
    out_type=jax.ShapeDtypeStruct((_NPAD, _FEAT), jnp.float32),
    mesh=_mesh,
    compiler_params=_cparams,
    scratch_types=[
        pltpu.VMEM((_NBLK, _EPB), jnp.int32),
        pltpu.VMEM((_NPAD,), jnp.float32),
        pltpu.VMEM((_NPAD,), jnp.float32),
        pltpu.VMEM((_NPW, _FEAT), jnp.float32),
        pltpu.VMEM((2, _EPB, _FEAT), jnp.float32),
        pltpu.VMEM((_GPB, _FEAT), jnp.float32),
        pltpu.SemaphoreType.DMA,
        pltpu.SemaphoreType.DMA,
    ],
)
def _agnn(x_hbm, col_hbm, inv_hbm, binv_hbm, out_hbm,
          colv, invv, binvv, xia, rows, outb, gsem, osem):
    wid = lax.axis_index("s") * 2 + lax.axis_index("c")
    pltpu.sync_copy(col_hbm.at[wid], colv)
    pltpu.sync_copy(inv_hbm, invv)
    pltpu.sync_copy(binv_hbm, binvv)
    pltpu.sync_copy(x_hbm.at[pl.ds(wid * _NPW, _NPW)], xia)
    lanes = lax.iota(jnp.int32, 16)
    pltpu.async_copy(x_hbm.at[colv.at[0]], rows.at[0], gsem)

    def block(b, carry):
        node0 = wid * _NPW + b * _GPB
        p = lax.rem(b, 2)
        pltpu.make_async_copy(x_hbm.at[colv.at[b]], rows.at[p], gsem).wait()

        @pl.when(b < _NBLK - 1)
        def _():
            pltpu.async_copy(x_hbm.at[colv.at[b + 1]], rows.at[1 - p], gsem)

        @pl.when(b > 0)
        def _():
            pltpu.make_async_copy(
                outb, out_hbm.at[pl.ds(node0 - _GPB, _GPB)], osem).wait()

        def node(n, c2):
            nb = n * _DEG
            xr = [xia[b * _GPB + n, pl.ds(r * 16, 16)] for r in range(8)]
            dots = []
            for g in range(2):
                d = jnp.zeros((16,), jnp.float32)
                for jj in range(16):
                    rr = nb + g * 16 + jj
                    acc = xr[0] * rows[p, rr, pl.ds(0, 16)]
                    for r in range(1, 8):
                        acc = acc + xr[r] * rows[p, rr, pl.ds(r * 16, 16)]
                    d = jnp.where(lanes == jj, _splat(jnp.sum(acc)), d)
                dots.append(d)
            bi = plsc.load_gather(binvv, [_splat(node0 + n, jnp.int32)])
            sc = []
            for g in range(2):
                jidx = colv[b, pl.ds(nb + g * 16, 16)]
                invj = plsc.load_gather(invv, [jidx])
                sc.append(dots[g] * invj * bi)
            m = _splat(jnp.max(jnp.maximum(sc[0], sc[1])))
            e0 = jnp.exp(sc[0] - m)
            e1 = jnp.exp(sc[1] - m)
            rden = 1.0 / (_splat(jnp.sum(e0 + e1)) + 1e-16)
            a0 = e0 * rden
            a1 = e1 * rden
            ao = [jnp.zeros((16,), jnp.float32) for _ in range(8)]
            for g, av in enumerate((a0, a1)):
                for jj in range(16):
                    a = av[jj]
                    rr = nb + g * 16 + jj
                    for r in range(8):
                        ao[r] = ao[r] + a * rows[p, rr, pl.ds(r * 16, 16)]
            for r in range(8):
                outb[n, pl.ds(r * 16, 16)] = ao[r]
            return c2

        lax.fori_loop(0, _GPB, node, 0)
        pltpu.async_copy(outb, out_hbm.at[pl.ds(node0, _GPB)], osem)
        return carry

    lax.fori_loop(0, _NBLK, block, 0)
    last0 = wid * _NPW + (_NBLK - 1) * _GPB
    pltpu.make_async_copy(outb, out_hbm.at[pl.ds(last0, _GPB)], osem).wait()


def kernel(x, row_id, row_ptr, col_id, beta):
    del row_id, row_ptr
    xp = jnp.zeros((_NPAD, _FEAT), jnp.float32).at[:_N].set(x)
    colp = jnp.zeros((_NPAD * _DEG,), jnp.int32).at[: _N * _DEG].set(col_id)
    col3 = colp.reshape(_NW, _NBLK, _EPB)
    beta16 = jnp.zeros((16,), jnp.float32).at[0].set(beta[0])
    inv, binv = _norms(xp, beta16)
    out = _agnn(xp, col3, inv, binv)
    return out[:_N]
